# Initial kernel scaffold; baseline (speedup 1.0000x reference)
import jax
import jax.numpy as jnp
from jax.experimental import pallas as pl


def kernel(x, edge_index, batch, W1, b1, W2, b2):
    raise NotImplementedError("write your pallas kernel here")


# speedup vs baseline: 10.6844x; 10.6844x over previous
import jax
import jax.numpy as jnp
from jax import lax
from jax.experimental import pallas as pl
from jax.experimental.pallas import tpu as pltpu
from jax.experimental.pallas import tpu_sc as plsc

N = 10000
E = 320000
D = 128
G = 64

NC = 2
NS = 16
NW = NC * NS
K = 128
CPW = 79
EPW = CPW * K
EP = NW * EPW
NPAD = 10240
RZ = NPAD // NS

R = 1000
GR = N // R


def _deg_body(dst_hbm, ones_hbm, zeros_hbm, out_hbm, idx_v, ones_v, acc):
    c = lax.axis_index("c")
    s = lax.axis_index("s")
    wid = c * NS + s
    r0 = s * RZ
    pltpu.sync_copy(zeros_hbm.at[pl.ds(r0, RZ)], acc.at[pl.ds(r0, RZ)])
    pltpu.sync_copy(ones_hbm, ones_v)
    plsc.subcore_barrier()
    base = wid * EPW

    @pl.loop(0, CPW)
    def _(i):
        pltpu.sync_copy(dst_hbm.at[pl.ds(base + i * K, K)], idx_v)
        pltpu.sync_copy(ones_v, acc.at[idx_v], add=True)

    plsc.subcore_barrier()
    pltpu.sync_copy(acc.at[pl.ds(r0, RZ)], out_hbm.at[c, pl.ds(r0, RZ)])


def _agg_body(src_hbm, dst_hbm, y_hbm, zeros_hbm, out_hbm,
              sidx, didx, rows, acc, sem):
    c = lax.axis_index("c")
    s = lax.axis_index("s")
    wid = c * NS + s
    r0 = s * RZ
    pltpu.sync_copy(zeros_hbm.at[pl.ds(r0, RZ)], acc.at[pl.ds(r0, RZ)])
    plsc.subcore_barrier()
    base = wid * EPW

    @pl.loop(0, CPW)
    def _(i):
        pltpu.sync_copy(src_hbm.at[pl.ds(base + i * K, K)], sidx)
        pltpu.sync_copy(dst_hbm.at[pl.ds(base + i * K, K)], didx)
        pltpu.async_copy(y_hbm.at[sidx], rows, sem).wait()
        pltpu.sync_copy(rows, acc.at[didx], add=True)

    plsc.subcore_barrier()
    pltpu.sync_copy(acc.at[pl.ds(r0, RZ)], out_hbm.at[c, pl.ds(r0, RZ)])


def _mm1_body(x_ref, w_ref, dinv_ref, o_ref):
    o_ref[...] = dinv_ref[...] * jnp.dot(
        x_ref[...], w_ref[...], preferred_element_type=jnp.float32)


def _mm2_body(z_ref, y1_ref, dinv_ref, b_ref, w_ref, o_ref):
    pre = dinv_ref[...] * (z_ref[0] + z_ref[1] + y1_ref[...]) + b_ref[...]
    h = jnp.maximum(pre, 0.0)
    o_ref[...] = dinv_ref[...] * jnp.dot(
        h, w_ref[...], preferred_element_type=jnp.float32)


def _pool_body(z_ref, y2_ref, dinv_ref, b_ref, batch_ref, o_ref, acc, cnt):
    i = pl.program_id(0)

    @pl.when(i == 0)
    def _():
        acc[...] = jnp.zeros_like(acc)
        cnt[...] = jnp.zeros_like(cnt)

    h = dinv_ref[...] * (z_ref[0] + z_ref[1] + y2_ref[...]) + b_ref[...]
    b_row = batch_ref[0]
    ids = lax.broadcasted_iota(jnp.int32, (128, R), 0)
    mask = (b_row == ids).astype(jnp.float32)
    acc[...] += lax.dot_general(
        mask, h, (((1,), (0,)), ((), ())), preferred_element_type=jnp.float32)
    cnt[...] += lax.dot_general(
        mask, jnp.ones((R, 1), jnp.float32), (((1,), (0,)), ((), ())),
        preferred_element_type=jnp.float32)

    @pl.when(i == GR - 1)
    def _():
        o_ref[...] = acc[:G, :] / jnp.maximum(cnt[:G, :], 1.0)


def _make_kernels(interpret=False):
    mesh = plsc.VectorSubcoreMesh(
        core_axis_name="c", subcore_axis_name="s",
        num_cores=NC, num_subcores=NS)

    deg = pl.kernel(
        _deg_body,
        out_type=jax.ShapeDtypeStruct((NC, NPAD, D), jnp.float32),
        mesh=mesh,
        scratch_types=[
            pltpu.VMEM((K,), jnp.int32),
            pltpu.VMEM((K, D), jnp.float32),
            pltpu.VMEM_SHARED((NPAD, D), jnp.float32),
        ],
        interpret=interpret,
    )

    agg = pl.kernel(
        _agg_body,
        out_type=jax.ShapeDtypeStruct((NC, NPAD, D), jnp.float32),
        mesh=mesh,
        scratch_types=[
            pltpu.VMEM((K,), jnp.int32),
            pltpu.VMEM((K,), jnp.int32),
            pltpu.VMEM((K, D), jnp.float32),
            pltpu.VMEM_SHARED((NPAD, D), jnp.float32),
            pltpu.SemaphoreType.DMA,
        ],
        interpret=interpret,
    )

    mm1 = pl.pallas_call(
        _mm1_body,
        grid=(GR,),
        in_specs=[
            pl.BlockSpec((R, D), lambda i: (i, 0)),
            pl.BlockSpec((D, D), lambda i: (0, 0)),
            pl.BlockSpec((R, 1), lambda i: (i, 0)),
        ],
        out_specs=pl.BlockSpec((R, D), lambda i: (i, 0)),
        out_shape=jax.ShapeDtypeStruct((N, D), jnp.float32),
        interpret=interpret,
    )

    mm2 = pl.pallas_call(
        _mm2_body,
        grid=(GR,),
        in_specs=[
            pl.BlockSpec((NC, R, D), lambda i: (0, i, 0)),
            pl.BlockSpec((R, D), lambda i: (i, 0)),
            pl.BlockSpec((R, 1), lambda i: (i, 0)),
            pl.BlockSpec((1, D), lambda i: (0, 0)),
            pl.BlockSpec((D, D), lambda i: (0, 0)),
        ],
        out_specs=pl.BlockSpec((R, D), lambda i: (i, 0)),
        out_shape=jax.ShapeDtypeStruct((N, D), jnp.float32),
        interpret=interpret,
    )

    pool = pl.pallas_call(
        _pool_body,
        grid=(GR,),
        in_specs=[
            pl.BlockSpec((NC, R, D), lambda i: (0, i, 0)),
            pl.BlockSpec((R, D), lambda i: (i, 0)),
            pl.BlockSpec((R, 1), lambda i: (i, 0)),
            pl.BlockSpec((1, D), lambda i: (0, 0)),
            pl.BlockSpec((1, 1, R), lambda i: (i, 0, 0)),
        ],
        out_specs=pl.BlockSpec((G, D), lambda i: (0, 0)),
        out_shape=jax.ShapeDtypeStruct((G, D), jnp.float32),
        scratch_shapes=[
            pltpu.VMEM((128, D), jnp.float32),
            pltpu.VMEM((128, 1), jnp.float32),
        ],
        interpret=interpret,
    )

    return deg, agg, mm1, mm2, pool


_DEG, _AGG, _MM1, _MM2, _POOL = _make_kernels()


def kernel(x, edge_index, batch, W1, b1, W2, b2):
    src = edge_index[0]
    dst = edge_index[1]
    padn = EP - E
    src_p = jnp.concatenate([src, jnp.zeros((padn,), jnp.int32)])
    dst_p = jnp.concatenate([dst, jnp.full((padn,), N, jnp.int32)])
    zeros128 = jnp.zeros((NPAD, D), jnp.float32)
    ones128 = jnp.ones((K, D), jnp.float32)

    degp = _DEG(dst_p, ones128, zeros128)
    deg = degp[0, :N, 0] + degp[1, :N, 0] + 1.0
    dinv = lax.rsqrt(deg).reshape(N, 1)

    y1 = _MM1(x, W1, dinv)
    zp1 = _AGG(src_p, dst_p, y1, zeros128)
    y2 = _MM2(zp1, y1, dinv, b1.reshape(1, D), W2)
    zp2 = _AGG(src_p, dst_p, y2, zeros128)
    return _POOL(zp2, y2, dinv, b2.reshape(1, D), batch.reshape(GR, 1, R))
